# baseline (speedup 1.0000x reference)
import jax
import jax.numpy as jnp
from jax import lax
from jax.experimental import pallas as pl
from jax.experimental.pallas import tpu as pltpu
from jax.experimental.pallas import tpu_sc as plsc

N_FEATURES = 26
INPUT_DIM = 33
OUT_DIM = 32
BATCH = 16384

NC, NS, L = 2, 16, 16
NW = NC * NS
TOTAL = BATCH * N_FEATURES
PER_W = TOTAL // NW
TAB_WORDS = N_FEATURES * INPUT_DIM * OUT_DIM
CHUNK = 1024
N_CHUNKS = PER_W // CHUNK
GROUPS = CHUNK // L
OFF_LEN = 208


def _embed_body(idx_hbm, off_hbm, tab_hbm, out_hbm,
                idx_v, off_v, tab_v, rows0, rows1, sw0, sw1):
    wid = lax.axis_index("s") * NC + lax.axis_index("c")
    wbase = wid * PER_W
    pltpu.sync_copy(off_hbm, off_v)
    pltpu.sync_copy(tab_hbm, tab_v)
    pltpu.sync_copy(idx_hbm.at[pl.ds(wbase, PER_W)], idx_v)

    def step(i, carry):
        off = off_v[pl.ds((i % (OFF_LEN // L)) * L, L)]
        idx_v[pl.ds(i * L, L)] = idx_v[pl.ds(i * L, L)] * OUT_DIM + off
        return carry

    lax.fori_loop(0, PER_W // L, step, 0)

    lane32 = lax.iota(jnp.int32, L) * OUT_DIM

    def chunk_compute(c, buf):
        def group(g, carry):
            a = idx_v[pl.ds(c * CHUNK + g * L, L)]
            d = lane32 + g * (L * OUT_DIM)
            for j in range(OUT_DIM):
                v = plsc.load_gather(tab_v, [a + j])
                plsc.store_scatter(buf, [d + j], v)
            return carry

        lax.fori_loop(0, GROUPS, group, 0)

    bufs = (rows0, rows1)
    wsems = (sw0, sw1)
    pend_w = [None, None]

    for c in range(N_CHUNKS):
        b = c % 2
        if pend_w[b] is not None:
            pend_w[b].wait()
        chunk_compute(c, bufs[b])
        wr = pltpu.make_async_copy(
            bufs[b],
            out_hbm.at[pl.ds((wbase + c * CHUNK) * OUT_DIM, CHUNK * OUT_DIM)],
            wsems[b],
        )
        wr.start()
        pend_w[b] = wr

    pend_w[(N_CHUNKS - 1) % 2].wait()
    pend_w[N_CHUNKS % 2].wait()


def kernel(inputs, tables):
    idx_flat = inputs.reshape(TOTAL)
    tab_flat = tables.reshape(TAB_WORDS)
    off = jnp.tile(
        jnp.arange(N_FEATURES, dtype=jnp.int32) * (INPUT_DIM * OUT_DIM),
        OFF_LEN // N_FEATURES,
    )

    run = pl.kernel(
        _embed_body,
        out_type=jax.ShapeDtypeStruct((TOTAL * OUT_DIM,), jnp.float32),
        mesh=plsc.VectorSubcoreMesh(core_axis_name="c", subcore_axis_name="s"),
        scratch_types=[
            pltpu.VMEM((PER_W,), jnp.int32),
            pltpu.VMEM((OFF_LEN,), jnp.int32),
            pltpu.VMEM((TAB_WORDS,), jnp.float32),
            pltpu.VMEM((CHUNK * OUT_DIM,), jnp.float32),
            pltpu.VMEM((CHUNK * OUT_DIM,), jnp.float32),
            pltpu.SemaphoreType.DMA,
            pltpu.SemaphoreType.DMA,
        ],
        compiler_params=pltpu.CompilerParams(
            use_tc_tiling_on_sc=False, needs_layout_passes=False
        ),
    )
    out = run(idx_flat, off, tab_flat)
    return out.reshape(BATCH, N_FEATURES * OUT_DIM)


# speedup vs baseline: 1.5051x; 1.5051x over previous
import jax
import jax.numpy as jnp
from jax import lax
from jax.experimental import pallas as pl
from jax.experimental.pallas import tpu as pltpu
from jax.experimental.pallas import tpu_sc as plsc

N_FEATURES = 26
INPUT_DIM = 33
OUT_DIM = 32
BATCH = 16384

NC, NS, L = 2, 16, 16
NW = NC * NS
TOTAL = BATCH * N_FEATURES
PER_W = TOTAL // NW
TAB_WORDS = N_FEATURES * INPUT_DIM * OUT_DIM
CHUNK = 1024
N_CHUNKS = PER_W // CHUNK
GROUPS = CHUNK // L
OFF_LEN = 208


def _embed_body(idx_hbm, off_hbm, tab_hbm, out_hbm,
                idx_v, off_v, tab_v, rows0, rows1, sw0, sw1):
    wid = lax.axis_index("s") * NC + lax.axis_index("c")
    wbase = wid * PER_W
    pltpu.sync_copy(off_hbm, off_v)
    pltpu.sync_copy(tab_hbm, tab_v)
    pltpu.sync_copy(idx_hbm.at[pl.ds(wbase, PER_W)], idx_v)

    @plsc.parallel_loop(0, PER_W // L)
    def _precompute(i):
        off = off_v[pl.ds((i % (OFF_LEN // L)) * L, L)]
        idx_v[pl.ds(i * L, L)] = idx_v[pl.ds(i * L, L)] * OUT_DIM + off

    lane32 = lax.iota(jnp.int32, L) * OUT_DIM

    def chunk_compute(c, buf):
        @plsc.parallel_loop(0, GROUPS)
        def _group(g):
            a = idx_v[pl.ds(c * CHUNK + g * L, L)]
            d = lane32 + g * (L * OUT_DIM)
            for j in range(OUT_DIM):
                v = plsc.load_gather(tab_v, [a + j])
                plsc.store_scatter(buf, [d + j], v)

    bufs = (rows0, rows1)
    wsems = (sw0, sw1)
    pend_w = [None, None]

    for c in range(N_CHUNKS):
        b = c % 2
        if pend_w[b] is not None:
            pend_w[b].wait()
        chunk_compute(c, bufs[b])
        wr = pltpu.make_async_copy(
            bufs[b],
            out_hbm.at[pl.ds((wbase + c * CHUNK) * OUT_DIM, CHUNK * OUT_DIM)],
            wsems[b],
        )
        wr.start()
        pend_w[b] = wr

    pend_w[(N_CHUNKS - 1) % 2].wait()
    pend_w[N_CHUNKS % 2].wait()


def kernel(inputs, tables):
    idx_flat = inputs.reshape(TOTAL)
    tab_flat = tables.reshape(TAB_WORDS)
    off = jnp.tile(
        jnp.arange(N_FEATURES, dtype=jnp.int32) * (INPUT_DIM * OUT_DIM),
        OFF_LEN // N_FEATURES,
    )

    run = pl.kernel(
        _embed_body,
        out_type=jax.ShapeDtypeStruct((TOTAL * OUT_DIM,), jnp.float32),
        mesh=plsc.VectorSubcoreMesh(core_axis_name="c", subcore_axis_name="s"),
        scratch_types=[
            pltpu.VMEM((PER_W,), jnp.int32),
            pltpu.VMEM((OFF_LEN,), jnp.int32),
            pltpu.VMEM((TAB_WORDS,), jnp.float32),
            pltpu.VMEM((CHUNK * OUT_DIM,), jnp.float32),
            pltpu.VMEM((CHUNK * OUT_DIM,), jnp.float32),
            pltpu.SemaphoreType.DMA,
            pltpu.SemaphoreType.DMA,
        ],
        compiler_params=pltpu.CompilerParams(
            use_tc_tiling_on_sc=False, needs_layout_passes=False
        ),
    )
    out = run(idx_flat, off, tab_flat)
    return out.reshape(BATCH, N_FEATURES * OUT_DIM)


# speedup vs baseline: 3.6331x; 2.4139x over previous
import jax
import jax.numpy as jnp
from jax import lax
from jax.experimental import pallas as pl
from jax.experimental.pallas import tpu as pltpu
from jax.experimental.pallas import tpu_sc as plsc

N_FEATURES = 26
INPUT_DIM = 33
OUT_DIM = 32
BATCH = 16384

NC, NS, L = 2, 16, 16
NW = NC * NS
TOTAL = BATCH * N_FEATURES
PER_W = TOTAL // NW
TAB_WORDS = N_FEATURES * INPUT_DIM * OUT_DIM
CHUNK = 1024
N_CHUNKS = PER_W // CHUNK
GROUPS = CHUNK // L
OFF_LEN = 208


def _embed_body(idx_hbm, off_hbm, tab_hbm, out_hbm,
                idx_v, off_v, tab_v, rows0, rows1, sw0, sw1):
    wid = lax.axis_index("s") * NC + lax.axis_index("c")
    wbase = wid * PER_W
    pltpu.sync_copy(off_hbm, off_v)
    pltpu.sync_copy(tab_hbm, tab_v)
    pltpu.sync_copy(idx_hbm.at[pl.ds(wbase, PER_W)], idx_v)

    @plsc.parallel_loop(0, PER_W // L)
    def _precompute(i):
        off = off_v[pl.ds((i % (OFF_LEN // L)) * L, L)]
        idx_v[pl.ds(i * L, L)] = idx_v[pl.ds(i * L, L)] * OUT_DIM + off

    def chunk_compute(c, buf):
        @plsc.parallel_loop(0, GROUPS, unroll=2)
        def _group(g):
            a16 = idx_v[pl.ds(c * CHUNK + g * L, L)]
            for k in range(L):
                a = a16[k]
                base = (g * L + k) * OUT_DIM
                buf[pl.ds(base, L)] = tab_v[pl.ds(a, L)]
                buf[pl.ds(base + L, L)] = tab_v[pl.ds(a + L, L)]

    bufs = (rows0, rows1)
    wsems = (sw0, sw1)
    pend_w = [None, None]

    for c in range(N_CHUNKS):
        b = c % 2
        if pend_w[b] is not None:
            pend_w[b].wait()
        chunk_compute(c, bufs[b])
        wr = pltpu.make_async_copy(
            bufs[b],
            out_hbm.at[pl.ds((wbase + c * CHUNK) * OUT_DIM, CHUNK * OUT_DIM)],
            wsems[b],
        )
        wr.start()
        pend_w[b] = wr

    pend_w[(N_CHUNKS - 1) % 2].wait()
    pend_w[N_CHUNKS % 2].wait()


def kernel(inputs, tables):
    idx_flat = inputs.reshape(TOTAL)
    tab_flat = tables.reshape(TAB_WORDS)
    off = jnp.tile(
        jnp.arange(N_FEATURES, dtype=jnp.int32) * (INPUT_DIM * OUT_DIM),
        OFF_LEN // N_FEATURES,
    )

    run = pl.kernel(
        _embed_body,
        out_type=jax.ShapeDtypeStruct((TOTAL * OUT_DIM,), jnp.float32),
        mesh=plsc.VectorSubcoreMesh(core_axis_name="c", subcore_axis_name="s"),
        scratch_types=[
            pltpu.VMEM((PER_W,), jnp.int32),
            pltpu.VMEM((OFF_LEN,), jnp.int32),
            pltpu.VMEM((TAB_WORDS,), jnp.float32),
            pltpu.VMEM((CHUNK * OUT_DIM,), jnp.float32),
            pltpu.VMEM((CHUNK * OUT_DIM,), jnp.float32),
            pltpu.SemaphoreType.DMA,
            pltpu.SemaphoreType.DMA,
        ],
        compiler_params=pltpu.CompilerParams(
            use_tc_tiling_on_sc=False, needs_layout_passes=False
        ),
    )
    out = run(idx_flat, off, tab_flat)
    return out.reshape(BATCH, N_FEATURES * OUT_DIM)


# speedup vs baseline: 3.6507x; 1.0048x over previous
import jax
import jax.numpy as jnp
from jax import lax
from jax.experimental import pallas as pl
from jax.experimental.pallas import tpu as pltpu
from jax.experimental.pallas import tpu_sc as plsc

N_FEATURES = 26
INPUT_DIM = 33
OUT_DIM = 32
BATCH = 16384

NC, NS, L = 2, 16, 16
NW = NC * NS
TOTAL = BATCH * N_FEATURES
PER_W = TOTAL // NW
TAB_WORDS = N_FEATURES * INPUT_DIM * OUT_DIM
CHUNK = 1024
N_CHUNKS = PER_W // CHUNK
GROUPS = CHUNK // L
OFF_LEN = 208


def _embed_body(idx_hbm, off_hbm, tab_hbm, out_hbm,
                idx_v, off_v, tab_v, rows0, rows1, idx_s, sw0, sw1):
    wid = lax.axis_index("s") * NC + lax.axis_index("c")
    wbase = wid * PER_W
    pltpu.sync_copy(off_hbm, off_v)
    pltpu.sync_copy(tab_hbm, tab_v)
    pltpu.sync_copy(idx_hbm.at[pl.ds(wbase, PER_W)], idx_v)

    @plsc.parallel_loop(0, PER_W // L)
    def _precompute(i):
        off = off_v[pl.ds((i % (OFF_LEN // L)) * L, L)]
        idx_v[pl.ds(i * L, L)] = idx_v[pl.ds(i * L, L)] * OUT_DIM + off

    def chunk_compute(c, buf, idx_s):
        @plsc.parallel_loop(0, GROUPS, unroll=4)
        def _group(g):
            a16 = idx_v[pl.ds(c * CHUNK + g * L, L)]
            for k in range(L):
                a = a16[k]
                base = (g * L + k) * OUT_DIM
                buf[pl.ds(base, L)] = tab_v[pl.ds(a, L)]
                buf[pl.ds(base + L, L)] = tab_v[pl.ds(a + L, L)]

    bufs = (rows0, rows1)
    wsems = (sw0, sw1)
    pend_w = [None, None]

    for c in range(N_CHUNKS):
        b = c % 2
        if pend_w[b] is not None:
            pend_w[b].wait()
        chunk_compute(c, bufs[b], idx_s)
        wr = pltpu.make_async_copy(
            bufs[b],
            out_hbm.at[pl.ds((wbase + c * CHUNK) * OUT_DIM, CHUNK * OUT_DIM)],
            wsems[b],
        )
        wr.start()
        pend_w[b] = wr

    pend_w[(N_CHUNKS - 1) % 2].wait()
    pend_w[N_CHUNKS % 2].wait()


def kernel(inputs, tables):
    idx_flat = inputs.reshape(TOTAL)
    tab_flat = tables.reshape(TAB_WORDS)
    off = jnp.tile(
        jnp.arange(N_FEATURES, dtype=jnp.int32) * (INPUT_DIM * OUT_DIM),
        OFF_LEN // N_FEATURES,
    )

    run = pl.kernel(
        _embed_body,
        out_type=jax.ShapeDtypeStruct((TOTAL * OUT_DIM,), jnp.float32),
        mesh=plsc.VectorSubcoreMesh(core_axis_name="c", subcore_axis_name="s"),
        scratch_types=[
            pltpu.VMEM((PER_W,), jnp.int32),
            pltpu.VMEM((OFF_LEN,), jnp.int32),
            pltpu.VMEM((TAB_WORDS,), jnp.float32),
            pltpu.VMEM((CHUNK * OUT_DIM,), jnp.float32),
            pltpu.VMEM((CHUNK * OUT_DIM,), jnp.float32),
            pltpu.SMEM((CHUNK,), jnp.int32),
            pltpu.SemaphoreType.DMA,
            pltpu.SemaphoreType.DMA,
        ],
        compiler_params=pltpu.CompilerParams(
            use_tc_tiling_on_sc=False, needs_layout_passes=False
        ),
    )
    out = run(idx_flat, off, tab_flat)
    return out.reshape(BATCH, N_FEATURES * OUT_DIM)


# speedup vs baseline: 3.6643x; 1.0037x over previous
import jax
import jax.numpy as jnp
from jax import lax
from jax.experimental import pallas as pl
from jax.experimental.pallas import tpu as pltpu
from jax.experimental.pallas import tpu_sc as plsc

N_FEATURES = 26
INPUT_DIM = 33
OUT_DIM = 32
BATCH = 16384

NC, NS, L = 2, 16, 16
NW = NC * NS
TOTAL = BATCH * N_FEATURES
PER_W = TOTAL // NW
TAB_WORDS = N_FEATURES * INPUT_DIM * OUT_DIM
CHUNK = 1024
N_CHUNKS = PER_W // CHUNK
GROUPS = CHUNK // L
OFF_LEN = 208


def _embed_body(idx_hbm, off_hbm, tab_hbm, out_hbm,
                idx_v, off_v, tab_v, rows0, rows1, idx_s, sw0, sw1):
    wid = lax.axis_index("s") * NC + lax.axis_index("c")
    wbase = wid * PER_W
    pltpu.sync_copy(off_hbm, off_v)
    pltpu.sync_copy(tab_hbm, tab_v)
    pltpu.sync_copy(idx_hbm.at[pl.ds(wbase, PER_W)], idx_v)

    @plsc.parallel_loop(0, PER_W // L)
    def _precompute(i):
        off = off_v[pl.ds((i % (OFF_LEN // L)) * L, L)]
        idx_v[pl.ds(i * L, L)] = idx_v[pl.ds(i * L, L)] * OUT_DIM + off

    def chunk_compute(c, buf, idx_s):
        @plsc.parallel_loop(0, GROUPS, unroll=4)
        def _group(g):
            a16 = idx_v[pl.ds(c * CHUNK + g * L, L)]
            for k in range(L):
                a = a16[k]
                base = (g * L + k) * OUT_DIM
                buf[pl.ds(base, L)] = tab_v[pl.ds(a, L)]
                buf[pl.ds(base + L, L)] = tab_v[pl.ds(a + L, L)]

    bufs = (rows0, rows1)
    wsems = (sw0, sw1)
    pend_w = [None, None]

    for c in range(N_CHUNKS):
        b = c % 2
        if pend_w[b] is not None:
            pend_w[b].wait()
        chunk_compute(c, bufs[b], idx_s)
        wr = pltpu.make_async_copy(
            bufs[b],
            out_hbm.at[pl.ds((wbase + c * CHUNK) * OUT_DIM, CHUNK * OUT_DIM)],
            wsems[b],
        )
        wr.start()
        pend_w[b] = wr

    pend_w[(N_CHUNKS - 1) % 2].wait()
    pend_w[N_CHUNKS % 2].wait()


def kernel(inputs, tables):
    idx_flat = inputs.reshape(TOTAL)
    tab_flat = tables.reshape(TAB_WORDS)
    off = jnp.tile(
        jnp.arange(N_FEATURES, dtype=jnp.int32) * (INPUT_DIM * OUT_DIM),
        OFF_LEN // N_FEATURES,
    )

    run = pl.kernel(
        _embed_body,
        out_type=jax.ShapeDtypeStruct((TOTAL * OUT_DIM,), jnp.float32),
        mesh=plsc.VectorSubcoreMesh(core_axis_name="c", subcore_axis_name="s"),
        scratch_types=[
            pltpu.VMEM((PER_W,), jnp.int32),
            pltpu.VMEM((OFF_LEN,), jnp.int32),
            pltpu.VMEM((TAB_WORDS,), jnp.float32),
            pltpu.VMEM((CHUNK * OUT_DIM,), jnp.float32),
            pltpu.VMEM((CHUNK * OUT_DIM,), jnp.float32),
            pltpu.SMEM((CHUNK,), jnp.int32),
            pltpu.SemaphoreType.DMA,
            pltpu.SemaphoreType.DMA,
        ],
        compiler_params=pltpu.CompilerParams(
            use_tc_tiling_on_sc=False,
            needs_layout_passes=False,
            disable_bounds_checks=True,
        ),
    )
    out = run(idx_flat, off, tab_flat)
    return out.reshape(BATCH, N_FEATURES * OUT_DIM)
